# Initial kernel scaffold; baseline (speedup 1.0000x reference)
import jax
import jax.numpy as jnp
from jax.experimental import pallas as pl


def kernel(idx, A):
    raise NotImplementedError("write your pallas kernel here")


# speedup vs baseline: 13.3493x; 13.3493x over previous
import functools

import jax
import jax.numpy as jnp
from jax.experimental import pallas as pl

_K = 32
_BR = 128


def _topk_mask_block(a_ref, o_ref):
    a = a_ref[...]
    v = jnp.maximum(a, 0.0)
    bits = jax.lax.bitcast_convert_type(v, jnp.int32)
    t = jnp.zeros((a.shape[0], 1), jnp.int32)
    for b in range(30, -1, -1):
        trial = t | (1 << b)
        cnt = jnp.sum((bits >= trial).astype(jnp.int32), axis=1, keepdims=True)
        t = jnp.where(cnt >= _K, trial, t)
    o_ref[...] = jnp.where(bits >= t, v, 0.0)


@jax.jit
def _run(A):
    n, m = A.shape
    grid = (n + _BR - 1) // _BR
    return pl.pallas_call(
        _topk_mask_block,
        grid=(grid,),
        in_specs=[pl.BlockSpec((_BR, m), lambda i: (i, 0))],
        out_specs=pl.BlockSpec((_BR, m), lambda i: (i, 0)),
        out_shape=jax.ShapeDtypeStruct((n, m), A.dtype),
    )(A)


def kernel(idx, A):
    return _run(A)
